# baseline (speedup 1.0000x reference)
import functools

import numpy as np

import jax
import jax.numpy as jnp
from jax import lax
from jax.experimental import pallas as pl
from jax.experimental.pallas import tpu as pltpu
from jax.experimental.pallas import tpu_sc as plsc

N = 10000
E = 320000
D = 128
L = 16
G = 64

NC = 2
NS = 16
CW = 125
CH = 80
KG = 16
KGX = 2
NP = 10240
PR = NP * L // 128
RPS = NP // NS

_MESH = plsc.VectorSubcoreMesh(
    core_axis_name="c", subcore_axis_name="s", num_cores=NC, num_subcores=NS)
_SC_PARAMS = pltpu.CompilerParams(use_tc_tiling_on_sc=False)
_BF = jnp.bfloat16
_F32 = jnp.float32


def _zero_fill(ref, nrows, ncols):
    def body(i, carry):
        for c0 in range(0, ncols, L):
            ref[i, pl.ds(c0, L)] = jnp.zeros((L,), _F32)
        return carry
    lax.fori_loop(0, nrows, body, None)


def _pipeline(table_hbm, src_v, dst_v, rows_v, acc, gsem, ssem, kg,
              extra_scatters):
    ngroups = CH // kg

    def fire_gathers(g):
        base, buf = g * kg, g % 2
        return [
            pltpu.async_copy(table_hbm.at[src_v.at[base + j]],
                             rows_v.at[buf, j], gsem[buf])
            for j in range(kg)
        ]

    def fire_scatters(g):
        base, buf = g * kg, g % 2
        puts = [
            pltpu.async_copy(rows_v.at[buf, j], acc.at[dst_v.at[base + j]],
                             ssem[buf], add=True)
            for j in range(kg)
        ]
        puts += extra_scatters(g, buf)
        return puts

    gd = {0: fire_gathers(0)}
    sd = {}
    for g in range(1, ngroups):
        if g >= 2:
            for d in sd[g - 2]:
                d.wait()
        gd[g] = fire_gathers(g)
        for d in gd[g - 1]:
            d.wait()
        sd[g - 1] = fire_scatters(g - 1)
    for d in gd[ngroups - 1]:
        d.wait()
    sd[ngroups - 1] = fire_scatters(ngroups - 1)
    for d in sd[ngroups - 2]:
        d.wait()
    for d in sd[ngroups - 1]:
        d.wait()


HD = D // 2


@functools.partial(
    pl.kernel,
    out_type=(jax.ShapeDtypeStruct((NC, NP, HD), _F32),
              jax.ShapeDtypeStruct((NC, NP, HD), _F32),
              jax.ShapeDtypeStruct((NC, NP, L), _F32)),
    mesh=_MESH,
    compiler_params=_SC_PARAMS,
    scratch_types=[
        pltpu.VMEM((CH, CW), jnp.int32),
        pltpu.VMEM((CH, CW), jnp.int32),
        pltpu.VMEM((2, KGX, CW, HD), _F32),
        pltpu.VMEM((CW, L), _F32),
        pltpu.VMEM((RPS // 4, HD), _F32),
        pltpu.VMEM((RPS, L), _F32),
        pltpu.VMEM_SHARED((NP, HD), _F32),
        pltpu.VMEM_SHARED((NP, L), _F32),
        pltpu.SemaphoreType.DMA,
        pltpu.SemaphoreType.DMA,
        pltpu.SemaphoreType.DMA,
        pltpu.SemaphoreType.DMA,
    ],
)
def _sc_aggx(xa_hbm, xb_hbm, edl_hbm, pa_hbm, pb_hbm, dparts_hbm,
             src_v, dst_v, rows_v, ones_v, zbufx, zbuf16, acc, dacc,
             gsem0, gsem1, ssem0, ssem1):
    c = lax.axis_index("c")
    s = lax.axis_index("s")

    def zero_acc():
        _zero_fill(zbufx, RPS // 4, HD)
        for kq in range(4):
            pltpu.sync_copy(zbufx, acc.at[pl.ds(s * RPS + kq * (RPS // 4),
                                                RPS // 4)])

    zero_acc()
    _zero_fill(zbuf16, RPS, L)
    pltpu.sync_copy(zbuf16, dacc.at[pl.ds(s * RPS, RPS)])

    def ones_body(i, carry):
        ones_v[i, :] = jnp.ones((L,), _F32)
        return carry
    lax.fori_loop(0, CW, ones_body, None)

    pltpu.sync_copy(edl_hbm.at[0, c, s], src_v)
    pltpu.sync_copy(edl_hbm.at[1, c, s], dst_v)
    plsc.subcore_barrier()

    def extra(g, buf):
        base = g * KGX
        return [
            pltpu.async_copy(ones_v, dacc.at[dst_v.at[base + j]],
                             (ssem0, ssem1)[buf], add=True)
            for j in range(KGX)
        ]

    _pipeline(xa_hbm, src_v, dst_v, rows_v, acc, (gsem0, gsem1),
              (ssem0, ssem1), KGX, extra)
    plsc.subcore_barrier()
    pltpu.sync_copy(acc.at[pl.ds(s * RPS, RPS)],
                    pa_hbm.at[c, pl.ds(s * RPS, RPS)])
    pltpu.sync_copy(dacc.at[pl.ds(s * RPS, RPS)],
                    dparts_hbm.at[c, pl.ds(s * RPS, RPS)])
    plsc.subcore_barrier()
    zero_acc()
    plsc.subcore_barrier()
    _pipeline(xb_hbm, src_v, dst_v, rows_v, acc, (gsem0, gsem1),
              (ssem0, ssem1), KGX, lambda g, buf: [])
    plsc.subcore_barrier()
    pltpu.sync_copy(acc.at[pl.ds(s * RPS, RPS)],
                    pb_hbm.at[c, pl.ds(s * RPS, RPS)])


@functools.partial(
    pl.kernel,
    out_type=jax.ShapeDtypeStruct((NC, NP, L), _F32),
    mesh=_MESH,
    compiler_params=_SC_PARAMS,
    scratch_types=[
        pltpu.VMEM((CH, CW), jnp.int32),
        pltpu.VMEM((CH, CW), jnp.int32),
        pltpu.VMEM((2, KG, CW, L), _F32),
        pltpu.VMEM((RPS, L), _F32),
        pltpu.VMEM_SHARED((NP, L), _F32),
        pltpu.SemaphoreType.DMA,
        pltpu.SemaphoreType.DMA,
        pltpu.SemaphoreType.DMA,
        pltpu.SemaphoreType.DMA,
    ],
)
def _sc_agg(h_hbm, edl_hbm, parts_hbm,
            src_v, dst_v, rows_v, zbuf, acc, gsem0, gsem1, ssem0, ssem1):
    c = lax.axis_index("c")
    s = lax.axis_index("s")
    _zero_fill(zbuf, RPS, L)
    pltpu.sync_copy(zbuf, acc.at[pl.ds(s * RPS, RPS)])
    pltpu.sync_copy(edl_hbm.at[0, c, s], src_v)
    pltpu.sync_copy(edl_hbm.at[1, c, s], dst_v)
    plsc.subcore_barrier()
    _pipeline(h_hbm, src_v, dst_v, rows_v, acc, (gsem0, gsem1),
              (ssem0, ssem1), KG, lambda g, buf: [])
    plsc.subcore_barrier()
    pltpu.sync_copy(acc.at[pl.ds(s * RPS, RPS)],
                    parts_hbm.at[c, pl.ds(s * RPS, RPS)])


def _bdot(a, b):
    return jnp.dot(a.astype(_BF), b.astype(_BF), preferred_element_type=_F32)


def _tc_combine0_body(pa_ref, pb_ref, deg_ref, x_ref, wl_ref, wr_ref, b_ref,
                      h_out):
    aggx = jnp.concatenate([pa_ref[0] + pa_ref[1], pb_ref[0] + pb_ref[1]],
                           axis=1)
    deg = deg_ref[0] + deg_ref[1]
    inv = 1.0 / jnp.maximum(deg[:, 0:1], 1.0)
    mean = aggx * inv
    a = _bdot(mean, wl_ref[...])
    b = _bdot(jnp.pad(x_ref[...], ((0, NP - N), (0, 0))), wr_ref[...])
    h_out[...] = jnp.maximum(a + b + b_ref[...], 0.0)


def _tc_combine0(pa, pb, deg, x, wl, wr, b):
    return pl.pallas_call(
        _tc_combine0_body,
        out_shape=jax.ShapeDtypeStruct((NP, L), _F32),
    )(pa, pb, deg, x, wl, wr, b.reshape(1, L))


def _tc_combine_body(parts_ref, dinv_ref, h_ref, b_ref, wl_ref, wr_ref,
                     h_out):
    mean = (parts_ref[0] + parts_ref[1]) * dinv_ref[...]
    a = _bdot(mean, wl_ref[...])
    b = _bdot(h_ref[...], wr_ref[...])
    h_out[...] = jnp.maximum(a + b + b_ref[...], 0.0)


def _tc_combine(parts, dinv, h, b8, wl8, wr8):
    return pl.pallas_call(
        _tc_combine_body,
        out_shape=jax.ShapeDtypeStruct((PR, 128), _F32),
    )(parts, dinv, h, b8, wl8, wr8)


def _tc_inv_body(dparts_ref, inv_out):
    deg = dparts_ref[0] + dparts_ref[1]
    inv_out[...] = 1.0 / jnp.maximum(deg, 1.0)


def _tc_inv(dparts):
    return pl.pallas_call(
        _tc_inv_body,
        out_shape=jax.ShapeDtypeStruct((PR, 128), _F32),
    )(dparts)


def _tc_final_body(h_ref, batb_ref, wro_ref, bro_ref, out_ref):
    h = h_ref[...]
    gids = lax.broadcasted_iota(jnp.int32, (G, PR), 0)
    pooled = jnp.zeros((G, L), _F32)
    for s in range(8):
        mask = (batb_ref[s:s + 1, :] == gids).astype(_F32)
        pooled = pooled + lax.dot_general(
            mask, h[:, s * L:(s + 1) * L], (((1,), (0,)), ((), ())),
            precision=lax.Precision.HIGHEST, preferred_element_type=_F32)
    z = _bdot(pooled, wro_ref[...]) + bro_ref[...]
    out_ref[...] = jax.nn.sigmoid(z)


def _tc_final(h, batb, wro, bro):
    return pl.pallas_call(
        _tc_final_body,
        out_shape=jax.ShapeDtypeStruct((G, 1), _F32),
    )(h, batb, wro, bro.reshape(1, 1))


def _kron8(w):
    return jnp.kron(jnp.eye(8, dtype=w.dtype), w)


def kernel(x, edge_index, batch, W_l0, W_r0, b0, W_l1, W_r1, b1,
           W_l2, W_r2, b2, W_ro, b_ro):
    edl = edge_index.reshape(2, NC, NS, CH, CW)

    pa, pb, dparts = _sc_aggx(x[:, :HD], x[:, HD:], edl)
    dv = dparts.reshape(NC, PR, 128)
    dinv = _tc_inv(dv)
    h1 = _tc_combine0(pa, pb, dparts, x, W_l0, W_r0, b0)

    parts1 = _sc_agg(h1, edl)
    h2 = _tc_combine(parts1.reshape(NC, PR, 128), dinv, h1.reshape(PR, 128),
                     jnp.tile(b1, 8).reshape(1, 128), _kron8(W_l1),
                     _kron8(W_r1))
    parts2 = _sc_agg(h2.reshape(NP, L), edl)
    h3 = _tc_combine(parts2.reshape(NC, PR, 128), dinv, h2,
                     jnp.tile(b2, 8).reshape(1, 128), _kron8(W_l2),
                     _kron8(W_r2))

    batb = jnp.pad(batch, (0, NP - N), constant_values=G).reshape(PR, 8).T
    return _tc_final(h3, batb, W_ro, b_ro)
